# baseline (speedup 1.0000x reference)
import functools

import jax
import jax.numpy as jnp
from jax import lax
from jax.experimental import pallas as pl
from jax.experimental.pallas import tpu as pltpu
from jax.experimental.pallas import tpu_sc as plsc

D = 128
L = 16
NC, NS = 2, 16
NW = NC * NS
B = 1024
S = 200
EPS = 1e-5
CH = 128
ROWS_PER_W = B * S // NW
NCHUNK = ROWS_PER_W // CH
NBUF = 3


def _rsqrt(x):
    bits = lax.bitcast_convert_type(x, jnp.int32)
    seed = lax.bitcast_convert_type(
        jnp.int32(0x5F3759DF) - lax.shift_right_logical(bits, 1), jnp.float32)
    y = seed
    for _ in range(3):
        y = y * (1.5 - 0.5 * x * y * y)
    return y


def _body(inp_hbm, table_hbm, pos_hbm, gamma_hbm, beta_hbm, out_hbm,
          idx_all, buf0, buf1, buf2, pos_v, gamma_v, beta_v,
          sg0, sg1, sg2, so0, so1, so2):
    cid = lax.axis_index("c")
    sid = lax.axis_index("s")
    wid = sid * NC + cid
    row0 = wid * ROWS_PER_W
    bufs = (buf0, buf1, buf2)
    sgs = (sg0, sg1, sg2)
    sos = (so0, so1, so2)

    pltpu.sync_copy(inp_hbm.at[wid], idx_all)
    pltpu.sync_copy(pos_hbm.at[pl.ds(0, S)], pos_v)
    pltpu.sync_copy(gamma_hbm, gamma_v)
    pltpu.sync_copy(beta_hbm, beta_v)

    gs = [gamma_v[pl.ds(L * j, L)] for j in range(D // L)]
    bs = [beta_v[pl.ds(L * j, L)] for j in range(D // L)]

    def compute(buf, base_row):
        @plsc.parallel_loop(0, CH, step=1, unroll=4)
        def s_body(j):
            s_pos = lax.rem(base_row + j, S)
            x = [buf[j, pl.ds(L * k, L)] + pos_v[s_pos, pl.ds(L * k, L)]
                 for k in range(D // L)]
            tot = ((x[0] + x[1]) + (x[2] + x[3])) + ((x[4] + x[5]) + (x[6] + x[7]))
            sq = [v * v for v in x]
            ssq = ((sq[0] + sq[1]) + (sq[2] + sq[3])) + ((sq[4] + sq[5]) + (sq[6] + sq[7]))
            mean = jnp.sum(tot) * (1.0 / D)
            var = jnp.sum(ssq) * (1.0 / D) - mean * mean
            mean_v = jnp.full((L,), mean, jnp.float32)
            rstd_v = _rsqrt(jnp.full((L,), var + EPS, jnp.float32))
            for k in range(D // L):
                buf[j, pl.ds(L * k, L)] = (x[k] - mean_v) * (rstd_v * gs[k]) + bs[k]

    def emit(i, slot, wait_prev_out, issue_next):
        nslot = (slot + 1) % NBUF
        if wait_prev_out:
            prev_off = pl.multiple_of(row0 + (i - 2) * CH, CH)
            pltpu.make_async_copy(
                bufs[nslot],
                out_hbm.at[pl.ds(prev_off, CH)],
                sos[nslot]).wait()
        if issue_next:
            pltpu.async_copy(table_hbm.at[idx_all.at[i + 1]], bufs[nslot],
                             sgs[nslot])
        pltpu.make_async_copy(table_hbm.at[idx_all.at[i]], bufs[slot],
                              sgs[slot]).wait()
        compute(bufs[slot], row0 + i * CH)
        cur_off = pl.multiple_of(row0 + i * CH, CH)
        pltpu.async_copy(bufs[slot],
                         out_hbm.at[pl.ds(cur_off, CH)], sos[slot])

    i0 = jnp.int32(0)
    pltpu.async_copy(table_hbm.at[idx_all.at[i0]], bufs[0], sgs[0])
    emit(i0, 0, False, True)
    emit(i0 + 1, 1, False, True)
    emit(i0 + 2, 2, True, True)

    def outer(g, c):
        for b in range(NBUF):
            emit(NBUF * g + b, b, True, True)
        return c

    lax.fori_loop(1, (NCHUNK - 2) // NBUF, outer, 0)

    emit(jnp.int32(NCHUNK - 2), (NCHUNK - 2) % NBUF, True, True)
    emit(jnp.int32(NCHUNK - 1), (NCHUNK - 1) % NBUF, True, False)

    for i in (NCHUNK - 2, NCHUNK - 1):
        slot = i % NBUF
        off = pl.multiple_of(row0 + i * CH, CH)
        pltpu.make_async_copy(bufs[slot],
                              out_hbm.at[pl.ds(off, CH)],
                              sos[slot]).wait()


@jax.jit
def _run(inp2d, word_table, pos_table, gamma, beta):
    mesh = plsc.VectorSubcoreMesh(core_axis_name="c", subcore_axis_name="s",
                                  num_cores=NC, num_subcores=NS)
    f = pl.kernel(
        _body,
        out_type=jax.ShapeDtypeStruct((B * S, D), jnp.float32),
        mesh=mesh,
        scratch_types=[
            pltpu.VMEM((NCHUNK, CH), jnp.int32),
            pltpu.VMEM((CH, D), jnp.float32),
            pltpu.VMEM((CH, D), jnp.float32),
            pltpu.VMEM((CH, D), jnp.float32),
            pltpu.VMEM((S, D), jnp.float32),
            pltpu.VMEM((D,), jnp.float32),
            pltpu.VMEM((D,), jnp.float32),
            pltpu.SemaphoreType.DMA,
            pltpu.SemaphoreType.DMA,
            pltpu.SemaphoreType.DMA,
            pltpu.SemaphoreType.DMA,
            pltpu.SemaphoreType.DMA,
            pltpu.SemaphoreType.DMA,
        ],
        compiler_params=pltpu.CompilerParams(needs_layout_passes=False),
    )
    return f(inp2d, word_table, pos_table, gamma, beta)


def kernel(inp, word_table, pos_table, gamma, beta):
    inp2d = inp.reshape(NW, NCHUNK, CH).astype(jnp.int32)
    out = _run(inp2d, word_table, pos_table, gamma, beta)
    return out.reshape(inp.shape[0], inp.shape[1], D)
